# baseline (speedup 1.0000x reference)
import functools

import jax
import jax.numpy as jnp
from jax import lax
from jax.experimental import pallas as pl
from jax.experimental.pallas import tpu as pltpu
from jax.experimental.pallas import tpu_sc as plsc

N = 10000
E = 320000
F_E = 16
N_GRAPHS = 16

NC = 2
NS = 16
NW = NC * NS
EDGES_PER_TILE = E // NW
BLK = 2000
NBLK = EDGES_PER_TILE // BLK
N_PAD = 10240
ROWS_PER_TILE = N_PAD // NS


def _sc_scatter_body(col_hbm, attr_hbm, sums_out, cnt_out,
                     attr0, attr1, idx0, idx1, ones_buf, sums_sh, cnt_sh,
                     isem0, isem1, jsem0, jsem1, ssem0, ssem1, csem0, csem1):
  c = lax.axis_index("c")
  s = lax.axis_index("s")
  wid = s * NC + c

  def init_row(i, _):
    attr0[i, :] = jnp.zeros((16,), jnp.float32)
    ones_buf[i, :] = jnp.ones((16,), jnp.float32)
    return 0
  lax.fori_loop(0, BLK, init_row, 0)

  row0 = s * ROWS_PER_TILE
  pltpu.sync_copy(attr0.at[pl.ds(0, ROWS_PER_TILE)],
                  sums_sh.at[pl.ds(row0, ROWS_PER_TILE)])
  pltpu.sync_copy(attr0.at[pl.ds(0, ROWS_PER_TILE)],
                  cnt_sh.at[pl.ds(row0, ROWS_PER_TILE)])
  plsc.subcore_barrier()

  base = wid * EDGES_PER_TILE
  attr = [attr0, attr1]
  idx = [idx0, idx1]
  isem = [isem0, isem1]
  jsem = [jsem0, jsem1]
  ssem = [ssem0, ssem1]
  csem = [csem0, csem1]

  def start_in(b, sl):
    off = base + b * BLK
    ha = pltpu.async_copy(attr_hbm.at[pl.ds(off, BLK)], attr[sl], isem[sl])
    hi = pltpu.async_copy(col_hbm.at[pl.ds(off, BLK)], idx[sl], jsem[sl])
    return ha, hi

  in_pending = [None, None]
  sc_pending = [None, None]
  in_pending[0] = start_in(0, 0)

  for b in range(NBLK):
    sl = b & 1
    ha, hi = in_pending[sl]
    ha.wait()
    hi.wait()
    in_pending[sl] = None
    if b + 1 < NBLK:
      if sc_pending[1 - sl] is not None:
        hs, hc = sc_pending[1 - sl]
        hs.wait()
        hc.wait()
        sc_pending[1 - sl] = None
      in_pending[1 - sl] = start_in(b + 1, 1 - sl)
    hs = pltpu.async_copy(attr[sl], sums_sh.at[idx[sl]], ssem[sl], add=True)
    hc = pltpu.async_copy(ones_buf, cnt_sh.at[idx[sl]], csem[sl], add=True)
    sc_pending[sl] = (hs, hc)

  for sl in (0, 1):
    if sc_pending[sl] is not None:
      hs, hc = sc_pending[sl]
      hs.wait()
      hc.wait()

  plsc.subcore_barrier()

  pltpu.sync_copy(sums_sh.at[pl.ds(row0, ROWS_PER_TILE)],
                  sums_out.at[c, pl.ds(row0, ROWS_PER_TILE)])
  pltpu.sync_copy(cnt_sh.at[pl.ds(row0, ROWS_PER_TILE)],
                  cnt_out.at[c, pl.ds(row0, ROWS_PER_TILE)])


def _sc_scatter(col, edge_attr_wide):
  mesh = plsc.VectorSubcoreMesh(core_axis_name="c", subcore_axis_name="s")
  kern = pl.kernel(
      _sc_scatter_body,
      out_type=[
          jax.ShapeDtypeStruct((NC, N_PAD, F_E), jnp.float32),
          jax.ShapeDtypeStruct((NC, N_PAD, F_E), jnp.float32),
      ],
      mesh=mesh,
      scratch_types=[
          pltpu.VMEM((BLK, F_E), jnp.float32),
          pltpu.VMEM((BLK, F_E), jnp.float32),
          pltpu.VMEM((BLK,), jnp.int32),
          pltpu.VMEM((BLK,), jnp.int32),
          pltpu.VMEM((BLK, F_E), jnp.float32),
          pltpu.VMEM_SHARED((N_PAD, F_E), jnp.float32),
          pltpu.VMEM_SHARED((N_PAD, F_E), jnp.float32),
          pltpu.SemaphoreType.DMA,
          pltpu.SemaphoreType.DMA,
          pltpu.SemaphoreType.DMA,
          pltpu.SemaphoreType.DMA,
          pltpu.SemaphoreType.DMA,
          pltpu.SemaphoreType.DMA,
          pltpu.SemaphoreType.DMA,
          pltpu.SemaphoreType.DMA,
      ],
      compiler_params=pltpu.CompilerParams(use_tc_tiling_on_sc=False,
                                           needs_layout_passes=False),
  )
  return kern(col, edge_attr_wide)


BN = 640


def _mlp_body(x_ref, s_ref, c_ref, batch_ref, u_ref,
              w1x_ref, w1e_ref, w1u_ref, b1_ref, w2_ref, b2_ref, out_ref):
  cnt = c_ref[0] + c_ref[1]
  e_agg = (s_ref[0] + s_ref[1]) / jnp.maximum(cnt, 1.0)
  uw = jnp.dot(u_ref[...], w1u_ref[...], preferred_element_type=jnp.float32)
  b = batch_ref[0, 0, :]
  onehot = jnp.where(
      b[:, None] == lax.broadcasted_iota(jnp.int32, (1, N_GRAPHS), 1),
      1.0, 0.0)
  h = jnp.dot(x_ref[...], w1x_ref[...], preferred_element_type=jnp.float32)
  h += jnp.dot(e_agg, w1e_ref[...], preferred_element_type=jnp.float32)
  h += jnp.dot(onehot, uw, preferred_element_type=jnp.float32)
  h = jnp.maximum(h + b1_ref[...], 0.0)
  out_ref[...] = jnp.dot(h, w2_ref[...],
                         preferred_element_type=jnp.float32) + b2_ref[...]


def _mlp(x, sums_p, cnt_p, batch3, u, w1x, w1e, w1u, b1, w2, b2):
  grid = N_PAD // BN
  full = lambda shape: pl.BlockSpec(shape, lambda i: (0,) * len(shape))
  return pl.pallas_call(
      _mlp_body,
      grid=(grid,),
      in_specs=[
          pl.BlockSpec((BN, 128), lambda i: (i, 0)),
          pl.BlockSpec((NC, BN, F_E), lambda i: (0, i, 0)),
          pl.BlockSpec((NC, BN, F_E), lambda i: (0, i, 0)),
          pl.BlockSpec((1, 1, BN), lambda i: (i, 0, 0)),
          full((N_GRAPHS, 128)),
          full((128, 128)),
          full((F_E, 128)),
          full((128, 128)),
          full((1, 128)),
          full((128, 128)),
          full((1, 128)),
      ],
      out_specs=pl.BlockSpec((BN, 128), lambda i: (i, 0)),
      out_shape=jax.ShapeDtypeStruct((N, 128), jnp.float32),
  )(x, sums_p, cnt_p, batch3, u, w1x, w1e, w1u, b1, w2, b2)


@jax.jit
def kernel(x, edge_index, edge_attr, u, batch, W1, b1, W2, b2):
  col = edge_index[1].astype(jnp.int32)
  sums_p, cnt_p = _sc_scatter(col, edge_attr)
  batch_pad = jnp.concatenate(
      [batch.astype(jnp.int32), jnp.zeros((N_PAD - N,), jnp.int32)])
  batch3 = batch_pad.reshape(N_PAD // BN, 1, BN)
  w1x = W1[:128]
  w1e = W1[128:128 + F_E]
  w1u = W1[128 + F_E:]
  return _mlp(x, sums_p, cnt_p, batch3, u,
              w1x, w1e, w1u, b1.reshape(1, 128), W2, b2.reshape(1, 128))


# speedup vs baseline: 1.0228x; 1.0228x over previous
import functools

import jax
import jax.numpy as jnp
from jax import lax
from jax.experimental import pallas as pl
from jax.experimental.pallas import tpu as pltpu
from jax.experimental.pallas import tpu_sc as plsc

N = 10000
E = 320000
F_E = 16
N_GRAPHS = 16

NC = 2
NS = 16
NW = NC * NS
EDGES_PER_TILE = E // NW
BLK = 1000
NBLK = EDGES_PER_TILE // BLK
N_PAD = 10240
ROWS_PER_TILE = N_PAD // NS


def _sc_scatter_body(col_hbm, attr_hbm, sums_out, cnt_out,
                     attr0, attr1, idx0, idx1, ones_buf, sums_sh, cnt_sh,
                     isem0, isem1, jsem0, jsem1, ssem0, ssem1, csem0, csem1):
  c = lax.axis_index("c")
  s = lax.axis_index("s")
  wid = s * NC + c

  def init_row(i, _):
    attr0[i, :] = jnp.zeros((16,), jnp.float32)
    ones_buf[i, :] = jnp.ones((16,), jnp.float32)
    return 0
  lax.fori_loop(0, BLK, init_row, 0)

  row0 = s * ROWS_PER_TILE
  pltpu.sync_copy(attr0.at[pl.ds(0, ROWS_PER_TILE)],
                  sums_sh.at[pl.ds(row0, ROWS_PER_TILE)])
  pltpu.sync_copy(attr0.at[pl.ds(0, ROWS_PER_TILE)],
                  cnt_sh.at[pl.ds(row0, ROWS_PER_TILE)])
  plsc.subcore_barrier()

  base = wid * EDGES_PER_TILE
  attr = [attr0, attr1]
  idx = [idx0, idx1]
  isem = [isem0, isem1]
  jsem = [jsem0, jsem1]
  ssem = [ssem0, ssem1]
  csem = [csem0, csem1]

  def start_in(b, sl):
    off = base + b * BLK
    ha = pltpu.async_copy(attr_hbm.at[pl.ds(off, BLK)], attr[sl], isem[sl])
    hi = pltpu.async_copy(col_hbm.at[pl.ds(off, BLK)], idx[sl], jsem[sl])
    return ha, hi

  in_pending = [None, None]
  sc_pending = [None, None]
  in_pending[0] = start_in(0, 0)

  for b in range(NBLK):
    sl = b & 1
    ha, hi = in_pending[sl]
    ha.wait()
    hi.wait()
    in_pending[sl] = None
    if b + 1 < NBLK:
      if sc_pending[1 - sl] is not None:
        hs, hc = sc_pending[1 - sl]
        hs.wait()
        hc.wait()
        sc_pending[1 - sl] = None
      in_pending[1 - sl] = start_in(b + 1, 1 - sl)
    hs = pltpu.async_copy(attr[sl], sums_sh.at[idx[sl]], ssem[sl], add=True)
    hc = pltpu.async_copy(ones_buf, cnt_sh.at[idx[sl]], csem[sl], add=True)
    sc_pending[sl] = (hs, hc)

  for sl in (0, 1):
    if sc_pending[sl] is not None:
      hs, hc = sc_pending[sl]
      hs.wait()
      hc.wait()

  plsc.subcore_barrier()

  pltpu.sync_copy(sums_sh.at[pl.ds(row0, ROWS_PER_TILE)],
                  sums_out.at[c, pl.ds(row0, ROWS_PER_TILE)])
  pltpu.sync_copy(cnt_sh.at[pl.ds(row0, ROWS_PER_TILE)],
                  cnt_out.at[c, pl.ds(row0, ROWS_PER_TILE)])


def _sc_scatter(col, edge_attr_wide):
  mesh = plsc.VectorSubcoreMesh(core_axis_name="c", subcore_axis_name="s")
  kern = pl.kernel(
      _sc_scatter_body,
      out_type=[
          jax.ShapeDtypeStruct((NC, N_PAD, F_E), jnp.float32),
          jax.ShapeDtypeStruct((NC, N_PAD, F_E), jnp.float32),
      ],
      mesh=mesh,
      scratch_types=[
          pltpu.VMEM((BLK, F_E), jnp.float32),
          pltpu.VMEM((BLK, F_E), jnp.float32),
          pltpu.VMEM((BLK,), jnp.int32),
          pltpu.VMEM((BLK,), jnp.int32),
          pltpu.VMEM((BLK, F_E), jnp.float32),
          pltpu.VMEM_SHARED((N_PAD, F_E), jnp.float32),
          pltpu.VMEM_SHARED((N_PAD, F_E), jnp.float32),
          pltpu.SemaphoreType.DMA,
          pltpu.SemaphoreType.DMA,
          pltpu.SemaphoreType.DMA,
          pltpu.SemaphoreType.DMA,
          pltpu.SemaphoreType.DMA,
          pltpu.SemaphoreType.DMA,
          pltpu.SemaphoreType.DMA,
          pltpu.SemaphoreType.DMA,
      ],
      compiler_params=pltpu.CompilerParams(use_tc_tiling_on_sc=False,
                                           needs_layout_passes=False,
                                           skip_device_barrier=True),
  )
  return kern(col, edge_attr_wide)


BN = 640


def _mlp_body(x_ref, s_ref, c_ref, batch_ref, u_ref,
              w1x_ref, w1e_ref, w1u_ref, b1_ref, w2_ref, b2_ref, out_ref):
  cnt = c_ref[0] + c_ref[1]
  e_agg = (s_ref[0] + s_ref[1]) / jnp.maximum(cnt, 1.0)
  uw = jnp.dot(u_ref[...], w1u_ref[...], preferred_element_type=jnp.float32)
  b = batch_ref[0, 0, :]
  onehot = jnp.where(
      b[:, None] == lax.broadcasted_iota(jnp.int32, (1, N_GRAPHS), 1),
      1.0, 0.0)
  h = jnp.dot(x_ref[...], w1x_ref[...], preferred_element_type=jnp.float32)
  h += jnp.dot(e_agg, w1e_ref[...], preferred_element_type=jnp.float32)
  h += jnp.dot(onehot, uw, preferred_element_type=jnp.float32)
  h = jnp.maximum(h + b1_ref[...], 0.0)
  out_ref[...] = jnp.dot(h, w2_ref[...],
                         preferred_element_type=jnp.float32) + b2_ref[...]


def _mlp(x, sums_p, cnt_p, batch3, u, w1x, w1e, w1u, b1, w2, b2):
  grid = N_PAD // BN
  full = lambda shape: pl.BlockSpec(shape, lambda i: (0,) * len(shape))
  return pl.pallas_call(
      _mlp_body,
      grid=(grid,),
      in_specs=[
          pl.BlockSpec((BN, 128), lambda i: (i, 0)),
          pl.BlockSpec((NC, BN, F_E), lambda i: (0, i, 0)),
          pl.BlockSpec((NC, BN, F_E), lambda i: (0, i, 0)),
          pl.BlockSpec((1, 1, BN), lambda i: (i, 0, 0)),
          full((N_GRAPHS, 128)),
          full((128, 128)),
          full((F_E, 128)),
          full((128, 128)),
          full((1, 128)),
          full((128, 128)),
          full((1, 128)),
      ],
      out_specs=pl.BlockSpec((BN, 128), lambda i: (i, 0)),
      out_shape=jax.ShapeDtypeStruct((N, 128), jnp.float32),
      compiler_params=pltpu.CompilerParams(skip_device_barrier=True),
  )(x, sums_p, cnt_p, batch3, u, w1x, w1e, w1u, b1, w2, b2)


@jax.jit
def kernel(x, edge_index, edge_attr, u, batch, W1, b1, W2, b2):
  col = edge_index[1].astype(jnp.int32)
  sums_p, cnt_p = _sc_scatter(col, edge_attr)
  batch_pad = jnp.concatenate(
      [batch.astype(jnp.int32), jnp.zeros((N_PAD - N,), jnp.int32)])
  batch3 = batch_pad.reshape(N_PAD // BN, 1, BN)
  w1x = W1[:128]
  w1e = W1[128:128 + F_E]
  w1u = W1[128 + F_E:]
  return _mlp(x, sums_p, cnt_p, batch3, u,
              w1x, w1e, w1u, b1.reshape(1, 128), W2, b2.reshape(1, 128))


# speedup vs baseline: 1.6770x; 1.6396x over previous
import functools

import jax
import jax.numpy as jnp
from jax import lax
from jax.experimental import pallas as pl
from jax.experimental.pallas import tpu as pltpu
from jax.experimental.pallas import tpu_sc as plsc

N = 10000
E = 320000
F_E = 16
N_GRAPHS = 16

NC = 2
NS = 16
CB = E // 128
CB_PER_TILE = CB // NS
KCB = 4
NSTEP = CB_PER_TILE // KCB
N_PAD = 10240


def _sc_scatter_body(col_hbm, attr4_hbm, sums_out, cnt_out,
                     a0, a1, i0, i1, sums8, cnt1,
                     sa0, sa1, si0, si1):
  c = lax.axis_index("c")
  s = lax.axis_index("s")

  def zrow(i, _):
    z = jnp.zeros((16,), jnp.float32)
    for f in range(8):
      sums8[f, pl.ds(i * 16, 16)] = z
    cnt1[pl.ds(i * 16, 16)] = z
    return 0
  lax.fori_loop(0, N_PAD // 16, zrow, 0)

  abuf = [a0, a1]
  ibuf = [i0, i1]
  asem = [sa0, sa1]
  isem = [si0, si1]
  cb0 = s * CB_PER_TILE
  ones16 = jnp.ones((16,), jnp.float32)

  def start_in(step, sl):
    cb = cb0 + step * KCB
    ha = pltpu.async_copy(attr4_hbm.at[c, pl.ds(cb, KCB)], abuf[sl], asem[sl])
    hi = pltpu.async_copy(col_hbm.at[pl.ds(cb * 128, KCB * 128)], ibuf[sl],
                          isem[sl])
    return ha, hi

  def consume(sl):
    def per_cb(cbl, _):
      def per_grp(g, _):
        lo = g * 16
        iv = ibuf[sl][pl.ds(cbl * 128 + lo, 16)]
        for f in range(8):
          v = abuf[sl][cbl, f, pl.ds(lo, 16)]
          plsc.addupdate_scatter(
              sums8, [jnp.full((16,), f, jnp.int32), iv], v)
        plsc.addupdate_scatter(cnt1, [iv], ones16)
        return 0
      lax.fori_loop(0, 8, per_grp, 0)
      return 0
    lax.fori_loop(0, KCB, per_cb, 0)

  pend = [None, None]
  pend[0] = start_in(0, 0)
  for step in range(NSTEP):
    sl = step & 1
    ha, hi = pend[sl]
    ha.wait()
    hi.wait()
    if step + 1 < NSTEP:
      pend[1 - sl] = start_in(step + 1, 1 - sl)
    consume(sl)

  @pl.when(s == NS - 1)
  def _tail():
    cb = NS * CB_PER_TILE
    pltpu.sync_copy(attr4_hbm.at[c, pl.ds(cb, KCB)], a0)
    pltpu.sync_copy(col_hbm.at[pl.ds(cb * 128, KCB * 128)], i0)
    consume(0)

  pltpu.sync_copy(sums8, sums_out.at[c, s])

  @pl.when(c == 0)
  def _wcnt():
    pltpu.sync_copy(cnt1, cnt_out.at[s])


def _sc_scatter(col, attr4):
  mesh = plsc.VectorSubcoreMesh(core_axis_name="c", subcore_axis_name="s")
  kern = pl.kernel(
      _sc_scatter_body,
      out_type=[
          jax.ShapeDtypeStruct((NC, NS, 8, N_PAD), jnp.float32),
          jax.ShapeDtypeStruct((NS, N_PAD), jnp.float32),
      ],
      mesh=mesh,
      scratch_types=[
          pltpu.VMEM((KCB, 8, 128), jnp.float32),
          pltpu.VMEM((KCB, 8, 128), jnp.float32),
          pltpu.VMEM((KCB * 128,), jnp.int32),
          pltpu.VMEM((KCB * 128,), jnp.int32),
          pltpu.VMEM((8, N_PAD), jnp.float32),
          pltpu.VMEM((N_PAD,), jnp.float32),
          pltpu.SemaphoreType.DMA,
          pltpu.SemaphoreType.DMA,
          pltpu.SemaphoreType.DMA,
          pltpu.SemaphoreType.DMA,
      ],
      compiler_params=pltpu.CompilerParams(use_tc_tiling_on_sc=False,
                                           needs_layout_passes=False,
                                           skip_device_barrier=True),
  )
  return kern(col, attr4)


BN = 640


def _mlp_body(x_ref, s_ref, c_ref, batch_ref, u_ref,
              w1x_ref, w1e_ref, w1u_ref, b1_ref, w2_ref, b2_ref, out_ref):
  ssum = jnp.sum(s_ref[...], axis=1)
  st = ssum.reshape(F_E, BN)
  cnt = jnp.sum(c_ref[...], axis=0)
  e_agg_t = st / jnp.maximum(cnt, 1.0)[None, :]
  uw = jnp.dot(u_ref[...], w1u_ref[...], preferred_element_type=jnp.float32)
  b = batch_ref[0, 0, :]
  onehot = jnp.where(
      b[:, None] == lax.broadcasted_iota(jnp.int32, (1, N_GRAPHS), 1),
      1.0, 0.0)
  h = jnp.dot(x_ref[...], w1x_ref[...], preferred_element_type=jnp.float32)
  h += jnp.dot(e_agg_t.T, w1e_ref[...], preferred_element_type=jnp.float32)
  h += jnp.dot(onehot, uw, preferred_element_type=jnp.float32)
  h = jnp.maximum(h + b1_ref[...], 0.0)
  out_ref[...] = jnp.dot(h, w2_ref[...],
                         preferred_element_type=jnp.float32) + b2_ref[...]


def _mlp(x, sums_p, cnt_p, batch3, u, w1x, w1e, w1u, b1, w2, b2):
  grid = N_PAD // BN
  full = lambda shape: pl.BlockSpec(shape, lambda i: (0,) * len(shape))
  return pl.pallas_call(
      _mlp_body,
      grid=(grid,),
      in_specs=[
          pl.BlockSpec((BN, 128), lambda i: (i, 0)),
          pl.BlockSpec((NC, NS, 8, BN), lambda i: (0, 0, 0, i)),
          pl.BlockSpec((NS, BN), lambda i: (0, i)),
          pl.BlockSpec((1, 1, BN), lambda i: (i, 0, 0)),
          full((N_GRAPHS, 128)),
          full((128, 128)),
          full((F_E, 128)),
          full((128, 128)),
          full((1, 128)),
          full((128, 128)),
          full((1, 128)),
      ],
      out_specs=pl.BlockSpec((BN, 128), lambda i: (i, 0)),
      out_shape=jax.ShapeDtypeStruct((N, 128), jnp.float32),
      compiler_params=pltpu.CompilerParams(skip_device_barrier=True),
  )(x, sums_p, cnt_p, batch3, u, w1x, w1e, w1u, b1, w2, b2)


@jax.jit
def kernel(x, edge_index, edge_attr, u, batch, W1, b1, W2, b2):
  col = edge_index[1].astype(jnp.int32)
  attr4 = edge_attr.T.reshape(2, 8, CB, 128).transpose(0, 2, 1, 3)
  sums_p, cnt_p = _sc_scatter(col, attr4)
  batch_pad = jnp.concatenate(
      [batch.astype(jnp.int32), jnp.zeros((N_PAD - N,), jnp.int32)])
  batch3 = batch_pad.reshape(N_PAD // BN, 1, BN)
  w1x = W1[:128]
  w1e = W1[128:128 + F_E]
  w1u = W1[128 + F_E:]
  return _mlp(x, sums_p, cnt_p, batch3, u,
              w1x, w1e, w1u, b1.reshape(1, 128), W2, b2.reshape(1, 128))


# speedup vs baseline: 1.7010x; 1.0143x over previous
import functools

import jax
import jax.numpy as jnp
from jax import lax
from jax.experimental import pallas as pl
from jax.experimental.pallas import tpu as pltpu
from jax.experimental.pallas import tpu_sc as plsc

N = 10000
E = 320000
F_E = 16
N_GRAPHS = 16

NC = 2
NS = 16
CB = E // 128
CB_PER_TILE = CB // NS
KCB = 12
NSTEP = CB_PER_TILE // KCB
KCBT = 4
N_PAD = 10240


def _sc_scatter_body(col_hbm, attr4_hbm, sums_out, cnt_out,
                     a0, a1, i0, i1, sums8, cnt1,
                     sa0, sa1, si0, si1):
  c = lax.axis_index("c")
  s = lax.axis_index("s")

  def zrow(i, _):
    z = jnp.zeros((16,), jnp.float32)
    for f in range(8):
      sums8[f, pl.ds(i * 16, 16)] = z
    cnt1[pl.ds(i * 16, 16)] = z
    return 0
  lax.fori_loop(0, N_PAD // 16, zrow, 0)

  abuf = [a0, a1]
  ibuf = [i0, i1]
  asem = [sa0, sa1]
  isem = [si0, si1]
  cb0 = s * CB_PER_TILE
  ones16 = jnp.ones((16,), jnp.float32)

  def start_in(step, sl):
    cb = cb0 + step * KCB
    ha = pltpu.async_copy(attr4_hbm.at[c, pl.ds(cb, KCB)], abuf[sl], asem[sl])
    hi = pltpu.async_copy(col_hbm.at[pl.ds(cb * 128, KCB * 128)], ibuf[sl],
                          isem[sl])
    return ha, hi

  def consume(sl, ncb=KCB):
    def per_cb(cbl, _):
      def per_grp(g, _):
        lo = g * 16
        iv = ibuf[sl][pl.ds(cbl * 128 + lo, 16)]
        for f in range(8):
          v = abuf[sl][cbl, f, pl.ds(lo, 16)]
          plsc.addupdate_scatter(
              sums8, [jnp.full((16,), f, jnp.int32), iv], v)
        plsc.addupdate_scatter(cnt1, [iv], ones16)
        return 0
      lax.fori_loop(0, 8, per_grp, 0)
      return 0
    lax.fori_loop(0, ncb, per_cb, 0)

  pend = [None, None]
  pend[0] = start_in(0, 0)
  for step in range(NSTEP):
    sl = step & 1
    ha, hi = pend[sl]
    ha.wait()
    hi.wait()
    if step + 1 < NSTEP:
      pend[1 - sl] = start_in(step + 1, 1 - sl)
    consume(sl)

  @pl.when(s == NS - 1)
  def _tail():
    cb = NS * CB_PER_TILE
    pltpu.sync_copy(attr4_hbm.at[c, pl.ds(cb, KCBT)], a0.at[pl.ds(0, KCBT)])
    pltpu.sync_copy(col_hbm.at[pl.ds(cb * 128, KCBT * 128)], i0.at[pl.ds(0, KCBT * 128)])
    consume(0, KCBT)

  pltpu.sync_copy(sums8, sums_out.at[c, s])

  @pl.when(c == 0)
  def _wcnt():
    pltpu.sync_copy(cnt1, cnt_out.at[s])


def _sc_scatter(col, attr4):
  mesh = plsc.VectorSubcoreMesh(core_axis_name="c", subcore_axis_name="s")
  kern = pl.kernel(
      _sc_scatter_body,
      out_type=[
          jax.ShapeDtypeStruct((NC, NS, 8, N_PAD), jnp.float32),
          jax.ShapeDtypeStruct((NS, N_PAD), jnp.float32),
      ],
      mesh=mesh,
      scratch_types=[
          pltpu.VMEM((KCB, 8, 128), jnp.float32),
          pltpu.VMEM((KCB, 8, 128), jnp.float32),
          pltpu.VMEM((KCB * 128,), jnp.int32),
          pltpu.VMEM((KCB * 128,), jnp.int32),
          pltpu.VMEM((8, N_PAD), jnp.float32),
          pltpu.VMEM((N_PAD,), jnp.float32),
          pltpu.SemaphoreType.DMA,
          pltpu.SemaphoreType.DMA,
          pltpu.SemaphoreType.DMA,
          pltpu.SemaphoreType.DMA,
      ],
      compiler_params=pltpu.CompilerParams(use_tc_tiling_on_sc=False,
                                           needs_layout_passes=False,
                                           skip_device_barrier=True),
  )
  return kern(col, attr4)


BN = 640


def _mlp_body(x_ref, s_ref, c_ref, batch_ref, u_ref,
              w1x_ref, w1e_ref, w1u_ref, b1_ref, w2_ref, b2_ref, out_ref):
  ssum = jnp.sum(s_ref[...], axis=1)
  st = ssum.reshape(F_E, BN)
  cnt = jnp.sum(c_ref[...], axis=0)
  e_agg_t = st / jnp.maximum(cnt, 1.0)[None, :]
  uw = jnp.dot(u_ref[...], w1u_ref[...], preferred_element_type=jnp.float32)
  b = batch_ref[0, 0, :]
  onehot = jnp.where(
      b[:, None] == lax.broadcasted_iota(jnp.int32, (1, N_GRAPHS), 1),
      1.0, 0.0)
  h = jnp.dot(x_ref[...], w1x_ref[...], preferred_element_type=jnp.float32)
  h += jnp.dot(e_agg_t.T, w1e_ref[...], preferred_element_type=jnp.float32)
  h += jnp.dot(onehot, uw, preferred_element_type=jnp.float32)
  h = jnp.maximum(h + b1_ref[...], 0.0)
  out_ref[...] = jnp.dot(h, w2_ref[...],
                         preferred_element_type=jnp.float32) + b2_ref[...]


def _mlp(x, sums_p, cnt_p, batch3, u, w1x, w1e, w1u, b1, w2, b2):
  grid = N_PAD // BN
  full = lambda shape: pl.BlockSpec(shape, lambda i: (0,) * len(shape))
  return pl.pallas_call(
      _mlp_body,
      grid=(grid,),
      in_specs=[
          pl.BlockSpec((BN, 128), lambda i: (i, 0)),
          pl.BlockSpec((NC, NS, 8, BN), lambda i: (0, 0, 0, i)),
          pl.BlockSpec((NS, BN), lambda i: (0, i)),
          pl.BlockSpec((1, 1, BN), lambda i: (i, 0, 0)),
          full((N_GRAPHS, 128)),
          full((128, 128)),
          full((F_E, 128)),
          full((128, 128)),
          full((1, 128)),
          full((128, 128)),
          full((1, 128)),
      ],
      out_specs=pl.BlockSpec((BN, 128), lambda i: (i, 0)),
      out_shape=jax.ShapeDtypeStruct((N, 128), jnp.float32),
      compiler_params=pltpu.CompilerParams(skip_device_barrier=True),
  )(x, sums_p, cnt_p, batch3, u, w1x, w1e, w1u, b1, w2, b2)


@jax.jit
def kernel(x, edge_index, edge_attr, u, batch, W1, b1, W2, b2):
  col = edge_index[1].astype(jnp.int32)
  attr4 = edge_attr.T.reshape(2, 8, CB, 128).transpose(0, 2, 1, 3)
  sums_p, cnt_p = _sc_scatter(col, attr4)
  batch_pad = jnp.concatenate(
      [batch.astype(jnp.int32), jnp.zeros((N_PAD - N,), jnp.int32)])
  batch3 = batch_pad.reshape(N_PAD // BN, 1, BN)
  w1x = W1[:128]
  w1e = W1[128:128 + F_E]
  w1u = W1[128 + F_E:]
  return _mlp(x, sums_p, cnt_p, batch3, u,
              w1x, w1e, w1u, b1.reshape(1, 128), W2, b2.reshape(1, 128))


# speedup vs baseline: 1.7156x; 1.0086x over previous
import functools

import jax
import jax.numpy as jnp
from jax import lax
from jax.experimental import pallas as pl
from jax.experimental.pallas import tpu as pltpu
from jax.experimental.pallas import tpu_sc as plsc

N = 10000
E = 320000
F_E = 16
N_GRAPHS = 16

NC = 2
NS = 16
CB = E // 128
CB_PER_TILE = CB // NS
KCB = 12
NSTEP = CB_PER_TILE // KCB
KCBT = 4
N_PAD = 10240


def _sc_scatter_body(col_hbm, attr4_hbm, sums_out, cnt_out,
                     a0, a1, i0, i1, sums8, cnt1,
                     sa0, sa1, si0, si1):
  c = lax.axis_index("c")
  s = lax.axis_index("s")

  def zrow(i, _):
    z = jnp.zeros((16,), jnp.float32)
    for f in range(8):
      sums8[f, pl.ds(i * 16, 16)] = z
    cnt1[pl.ds(i * 16, 16)] = z
    return 0
  lax.fori_loop(0, N_PAD // 16, zrow, 0)

  abuf = [a0, a1]
  ibuf = [i0, i1]
  asem = [sa0, sa1]
  isem = [si0, si1]
  cb0 = s * CB_PER_TILE
  ones16 = jnp.ones((16,), jnp.float32)

  def start_in(step, sl):
    cb = cb0 + step * KCB
    ha = pltpu.async_copy(attr4_hbm.at[c, pl.ds(cb, KCB)], abuf[sl], asem[sl])
    hi = pltpu.async_copy(col_hbm.at[pl.ds(cb * 128, KCB * 128)], ibuf[sl],
                          isem[sl])
    return ha, hi

  def consume(sl, ncb=KCB):
    def per_cb(cbl, _):
      def per_grp(g, _):
        lo = g * 16
        iv = ibuf[sl][pl.ds(cbl * 128 + lo, 16)]
        for f in range(8):
          v = abuf[sl][cbl, f, pl.ds(lo, 16)]
          plsc.addupdate_scatter(
              sums8, [jnp.full((16,), f, jnp.int32), iv], v)
        plsc.addupdate_scatter(cnt1, [iv], ones16)
        return 0
      lax.fori_loop(0, 8, per_grp, 0)
      return 0
    lax.fori_loop(0, ncb, per_cb, 0)

  pend = [None, None]
  pend[0] = start_in(0, 0)
  for step in range(NSTEP):
    sl = step & 1
    ha, hi = pend[sl]
    ha.wait()
    hi.wait()
    if step + 1 < NSTEP:
      pend[1 - sl] = start_in(step + 1, 1 - sl)
    consume(sl)

  @pl.when(s == NS - 1)
  def _tail():
    cb = NS * CB_PER_TILE
    pltpu.sync_copy(attr4_hbm.at[c, pl.ds(cb, KCBT)], a0.at[pl.ds(0, KCBT)])
    pltpu.sync_copy(col_hbm.at[pl.ds(cb * 128, KCBT * 128)], i0.at[pl.ds(0, KCBT * 128)])
    consume(0, KCBT)

  pltpu.sync_copy(sums8, sums_out.at[c, s])

  @pl.when(c == 0)
  def _wcnt():
    pltpu.sync_copy(cnt1, cnt_out.at[s])


def _sc_scatter(col, attr4):
  mesh = plsc.VectorSubcoreMesh(core_axis_name="c", subcore_axis_name="s")
  kern = pl.kernel(
      _sc_scatter_body,
      out_type=[
          jax.ShapeDtypeStruct((NC, NS, 8, N_PAD), jnp.float32),
          jax.ShapeDtypeStruct((NS, N_PAD), jnp.float32),
      ],
      mesh=mesh,
      scratch_types=[
          pltpu.VMEM((KCB, 8, 128), jnp.float32),
          pltpu.VMEM((KCB, 8, 128), jnp.float32),
          pltpu.VMEM((KCB * 128,), jnp.int32),
          pltpu.VMEM((KCB * 128,), jnp.int32),
          pltpu.VMEM((8, N_PAD), jnp.float32),
          pltpu.VMEM((N_PAD,), jnp.float32),
          pltpu.SemaphoreType.DMA,
          pltpu.SemaphoreType.DMA,
          pltpu.SemaphoreType.DMA,
          pltpu.SemaphoreType.DMA,
      ],
      compiler_params=pltpu.CompilerParams(use_tc_tiling_on_sc=False,
                                           needs_layout_passes=False,
                                           skip_device_barrier=True),
  )
  return kern(col, attr4)


BN = 640


def _mlp_pre_body(x_ref, batch_ref, u_ref, w1x_ref, w1u_ref, b1_ref, hx_ref):
  uw = jnp.dot(u_ref[...], w1u_ref[...], preferred_element_type=jnp.float32)
  b = batch_ref[0, 0, :]
  onehot = jnp.where(
      b[:, None] == lax.broadcasted_iota(jnp.int32, (1, N_GRAPHS), 1),
      1.0, 0.0)
  h = jnp.dot(x_ref[...], w1x_ref[...], preferred_element_type=jnp.float32)
  h += jnp.dot(onehot, uw, preferred_element_type=jnp.float32)
  hx_ref[...] = h + b1_ref[...]


def _mlp_pre(x, batch3, u, w1x, w1u, b1):
  grid = N_PAD // BN
  full = lambda shape: pl.BlockSpec(shape, lambda i: (0,) * len(shape))
  return pl.pallas_call(
      _mlp_pre_body,
      grid=(grid,),
      in_specs=[
          pl.BlockSpec((BN, 128), lambda i: (i, 0)),
          pl.BlockSpec((1, 1, BN), lambda i: (i, 0, 0)),
          full((N_GRAPHS, 128)),
          full((128, 128)),
          full((128, 128)),
          full((1, 128)),
      ],
      out_specs=pl.BlockSpec((BN, 128), lambda i: (i, 0)),
      out_shape=jax.ShapeDtypeStruct((N, 128), jnp.float32),
      compiler_params=pltpu.CompilerParams(skip_device_barrier=True),
  )(x, batch3, u, w1x, w1u, b1)


def _mlp_post_body(hx_ref, s_ref, c_ref, w1e_ref, w2_ref, b2_ref, out_ref):
  ssum = jnp.sum(s_ref[...], axis=1)
  st = ssum.reshape(F_E, BN)
  cnt = jnp.sum(c_ref[...], axis=0)
  e_agg_t = st / jnp.maximum(cnt, 1.0)[None, :]
  h = hx_ref[...] + jnp.dot(e_agg_t.T, w1e_ref[...],
                            preferred_element_type=jnp.float32)
  h = jnp.maximum(h, 0.0)
  out_ref[...] = jnp.dot(h, w2_ref[...],
                         preferred_element_type=jnp.float32) + b2_ref[...]


def _mlp_post(hx, sums_p, cnt_p, w1e, w2, b2):
  grid = N_PAD // BN
  full = lambda shape: pl.BlockSpec(shape, lambda i: (0,) * len(shape))
  return pl.pallas_call(
      _mlp_post_body,
      grid=(grid,),
      in_specs=[
          pl.BlockSpec((BN, 128), lambda i: (i, 0)),
          pl.BlockSpec((NC, NS, 8, BN), lambda i: (0, 0, 0, i)),
          pl.BlockSpec((NS, BN), lambda i: (0, i)),
          full((F_E, 128)),
          full((128, 128)),
          full((1, 128)),
      ],
      out_specs=pl.BlockSpec((BN, 128), lambda i: (i, 0)),
      out_shape=jax.ShapeDtypeStruct((N, 128), jnp.float32),
      compiler_params=pltpu.CompilerParams(skip_device_barrier=True),
  )(hx, sums_p, cnt_p, w1e, w2, b2)


@jax.jit
def kernel(x, edge_index, edge_attr, u, batch, W1, b1, W2, b2):
  col = edge_index[1].astype(jnp.int32)
  attr4 = edge_attr.T.reshape(2, 8, CB, 128).transpose(0, 2, 1, 3)
  sums_p, cnt_p = _sc_scatter(col, attr4)
  batch_pad = jnp.concatenate(
      [batch.astype(jnp.int32), jnp.zeros((N_PAD - N,), jnp.int32)])
  batch3 = batch_pad.reshape(N_PAD // BN, 1, BN)
  w1x = W1[:128]
  w1e = W1[128:128 + F_E]
  w1u = W1[128 + F_E:]
  hx = _mlp_pre(x, batch3, u, w1x, w1u, b1.reshape(1, 128))
  return _mlp_post(hx, sums_p, cnt_p, w1e, W2, b2.reshape(1, 128))
